# baseline (speedup 1.0000x reference)
import functools

import jax
import jax.numpy as jnp
import numpy as np
from jax import lax
from jax.experimental import pallas as pl
from jax.experimental.pallas import tpu as pltpu
from jax.experimental.pallas import tpu_sc as plsc

N = 10000
E = 320000
IN_DIM = 128
HD = 32
NH = 4
D = NH * HD
DW = D + 16

NC, NS = 2, 16
NW = NC * NS
EPW = E // NW
C = 32
NPAIR = 156
TAIL = EPW - 2 * NPAIR * C
NP = 10240
RPT = NP // NS

_ROW_BLK = 1000


def _qkv_body(h_ref, w_ref, qv_ref, k_ref):
    r = jnp.dot(h_ref[...], w_ref[...], preferred_element_type=jnp.float32)
    qv_ref[...] = jnp.concatenate([r[:, :D], r[:, 2 * D:]], axis=1)
    k_ref[...] = r[:, D:2 * D]


def _norm_body(x0_ref, x1_ref, mexp_ref, o_ref):
    x = x0_ref[...] + x1_ref[...]
    p = x[:, :D]
    d = x[:, D:]
    dc = jnp.dot(d, mexp_ref[...], preferred_element_type=jnp.float32)
    o_ref[...] = jnp.where(dc > 0, p / dc, 0.0)


def _deinterleave(eidx, src_ref, dst_ref, n):
    lanes0 = lax.iota(jnp.int32, 16)
    zc = jnp.zeros((16,), jnp.int32)
    oc = zc + 1
    for j in range(n // 16):
        lanes = lanes0 + j * 16
        src_ref[pl.ds(j * 16, 16)] = plsc.load_gather(eidx, [lanes, zc])
        dst_ref[pl.ds(j * 16, 16)] = plsc.load_gather(eidx, [lanes, oc])


def _compute_chunk(qvb, kb, wb, ngroups):
    zeros16 = jnp.zeros((16,), jnp.float32)
    lanes0 = lax.iota(jnp.int32, 16)
    UNR = 4
    for g in range(ngroups):
        lanes = lanes0 + (g * 16)

        def dot_body(cb, accs):
            a = list(accs)
            cbase = lanes0 + cb * UNR
            for u in range(UNR):
                rot = (cbase + u) & (HD - 1)
                for h in range(NH):
                    col = rot + (h * HD)
                    qc = plsc.load_gather(qvb, [lanes, col])
                    kc = plsc.load_gather(kb, [lanes, col])
                    a[h] = a[h] + qc * kc
            return tuple(a)
        acc = lax.fori_loop(0, HD // UNR, dot_body, (zeros16,) * NH)

        ss = []
        for h in range(NH):
            e = acc[h]
            e = jnp.where(e < 0, e * 0.2, e)
            s = jnp.exp(e)
            plsc.store_scatter(wb, [lanes, jnp.full((16,), D + h, jnp.int32)], s)
            ss.append(s)

        def scale_body(cb, carry):
            cbase = lanes0 + cb * UNR
            for u in range(UNR):
                rot = (cbase + u) & (HD - 1)
                for h in range(NH):
                    col = rot + (h * HD)
                    vc = plsc.load_gather(qvb, [lanes, D + col])
                    plsc.store_scatter(wb, [lanes, col], vc * ss[h])
            return carry
        lax.fori_loop(0, HD // UNR, scale_body, 0)


def _copy_idx(dst_ref, src_ref):
    for j in range(C // 16):
        dst_ref[pl.ds(j * 16, 16)] = src_ref[pl.ds(j * 16, 16)]


def _edge_kernel_body(qv_hbm, k_hbm, eidx_hbm, outp,
                      eidxA, srcA, dstA, dscA, qvbA, kbA, wbA,
                      eidxB, srcB, dstB, dscB, qvbB, kbB, wbB,
                      eidxT, srcT, dstT,
                      acc_sp,
                      gsemA, gsemB, ssemA, ssemB, isemA, isemB):
    cid = lax.axis_index("c")
    sid = lax.axis_index("s")
    zeros16 = jnp.zeros((16,), jnp.float32)

    def zrow(r, _):
        for j in range(DW // 16):
            wbA[r, pl.ds(j * 16, 16)] = zeros16
        wbB[r, pl.ds(D + 16 - 16, 16)] = zeros16
        return 0
    lax.fori_loop(0, C, zrow, 0)

    for b in range(RPT // C):
        pltpu.sync_copy(wbA, acc_sp.at[pl.ds(sid * RPT + b * C, C)])

    plsc.subcore_barrier()

    ebase = (cid * NS + sid) * EPW

    pltpu.sync_copy(eidx_hbm.at[pl.ds(ebase, C)], eidxA)
    _deinterleave(eidxA, srcA, dstA, C)
    pltpu.async_copy(qv_hbm.at[srcA], qvbA, gsemA)
    pltpu.async_copy(k_hbm.at[dstA], kbA, gsemA)

    def pair_body(t, _):
        baseB = ebase + (2 * t + 1) * C
        baseA2 = ebase + (2 * t + 2) * C

        pltpu.async_copy(eidx_hbm.at[pl.ds(baseB, C)], eidxB, isemB)

        @pl.when(t > 0)
        def _():
            pltpu.make_async_copy(wbA, acc_sp.at[dscA], ssemA).wait()

        pltpu.make_async_copy(qv_hbm.at[srcA], qvbA, gsemA).wait()
        pltpu.make_async_copy(k_hbm.at[dstA], kbA, gsemA).wait()
        _copy_idx(dscA, dstA)

        @pl.when(t < NPAIR - 1)
        def _():
            pltpu.async_copy(eidx_hbm.at[pl.ds(baseA2, C)], eidxA, isemA)

        pltpu.make_async_copy(eidx_hbm.at[pl.ds(baseB, C)], eidxB,
                              isemB).wait()
        _deinterleave(eidxB, srcB, dstB, C)
        pltpu.async_copy(qv_hbm.at[srcB], qvbB, gsemB)
        pltpu.async_copy(k_hbm.at[dstB], kbB, gsemB)

        _compute_chunk(qvbA, kbA, wbA, C // 16)
        pltpu.async_copy(wbA, acc_sp.at[dscA], ssemA, add=True)

        @pl.when(t < NPAIR - 1)
        def _():
            pltpu.make_async_copy(eidx_hbm.at[pl.ds(baseA2, C)], eidxA,
                                  isemA).wait()
            _deinterleave(eidxA, srcA, dstA, C)
            pltpu.async_copy(qv_hbm.at[srcA], qvbA, gsemA)
            pltpu.async_copy(k_hbm.at[dstA], kbA, gsemA)

        @pl.when(t > 0)
        def _():
            pltpu.make_async_copy(wbB, acc_sp.at[dscB], ssemB).wait()

        pltpu.make_async_copy(qv_hbm.at[srcB], qvbB, gsemB).wait()
        pltpu.make_async_copy(k_hbm.at[dstB], kbB, gsemB).wait()
        _copy_idx(dscB, dstB)
        _compute_chunk(qvbB, kbB, wbB, C // 16)
        pltpu.async_copy(wbB, acc_sp.at[dscB], ssemB, add=True)
        return 0

    lax.fori_loop(0, NPAIR, pair_body, 0)

    pltpu.make_async_copy(wbA, acc_sp.at[dscA], ssemA).wait()
    pltpu.make_async_copy(wbB, acc_sp.at[dscB], ssemB).wait()

    tbase = ebase + 2 * NPAIR * C
    pltpu.sync_copy(eidx_hbm.at[pl.ds(tbase, TAIL)], eidxT)
    _deinterleave(eidxT, srcT, dstT, TAIL)
    pltpu.sync_copy(qv_hbm.at[srcT], qvbA.at[pl.ds(0, TAIL)])
    pltpu.sync_copy(k_hbm.at[dstT], kbA.at[pl.ds(0, TAIL)])
    _compute_chunk(qvbA, kbA, wbA, TAIL // 16)
    pltpu.sync_copy(wbA.at[pl.ds(0, TAIL)], acc_sp.at[dstT], add=True)

    plsc.subcore_barrier()

    r0 = sid * RPT
    pltpu.sync_copy(acc_sp.at[pl.ds(r0, RPT)], outp.at[pl.ds(cid * NP + r0, RPT)])


_edge_kernel = functools.partial(
    pl.kernel,
    out_type=jax.ShapeDtypeStruct((NC * NP, DW), jnp.float32),
    mesh=plsc.VectorSubcoreMesh(core_axis_name="c", subcore_axis_name="s"),
    compiler_params=pltpu.CompilerParams(use_tc_tiling_on_sc=False,
                                         needs_layout_passes=False),
    scratch_types=(
        pltpu.VMEM((C, 2), jnp.int32),
        pltpu.VMEM((C,), jnp.int32),
        pltpu.VMEM((C,), jnp.int32),
        pltpu.VMEM((C,), jnp.int32),
        pltpu.VMEM((C, 2 * D), jnp.float32),
        pltpu.VMEM((C, D), jnp.float32),
        pltpu.VMEM((C, DW), jnp.float32),
        pltpu.VMEM((C, 2), jnp.int32),
        pltpu.VMEM((C,), jnp.int32),
        pltpu.VMEM((C,), jnp.int32),
        pltpu.VMEM((C,), jnp.int32),
        pltpu.VMEM((C, 2 * D), jnp.float32),
        pltpu.VMEM((C, D), jnp.float32),
        pltpu.VMEM((C, DW), jnp.float32),
        pltpu.VMEM((TAIL, 2), jnp.int32),
        pltpu.VMEM((TAIL,), jnp.int32),
        pltpu.VMEM((TAIL,), jnp.int32),
        pltpu.VMEM_SHARED((NP, DW), jnp.float32),
        pltpu.SemaphoreType.DMA,
        pltpu.SemaphoreType.DMA,
        pltpu.SemaphoreType.DMA,
        pltpu.SemaphoreType.DMA,
        pltpu.SemaphoreType.DMA,
        pltpu.SemaphoreType.DMA,
    ),
)(_edge_kernel_body)


_MEXP = np.zeros((16, D), np.float32)
for _h in range(NH):
    _MEXP[_h, _h * HD:(_h + 1) * HD] = 1.0


def kernel(h, edge_index, Wq, Wk, Wv):
    h = h.astype(jnp.float32)
    eidx = edge_index.astype(jnp.int32).T
    wq = jnp.transpose(Wq, (1, 0, 2)).reshape(IN_DIM, D)
    wk = jnp.transpose(Wk, (1, 0, 2)).reshape(IN_DIM, D)
    wv = jnp.transpose(Wv, (1, 0, 2)).reshape(IN_DIM, D)
    w3 = jnp.concatenate([wq, wk, wv], axis=1)

    qv, kk = pl.pallas_call(
        _qkv_body,
        grid=(N // _ROW_BLK,),
        in_specs=[pl.BlockSpec((_ROW_BLK, IN_DIM), lambda i: (i, 0)),
                  pl.BlockSpec((IN_DIM, 3 * D), lambda i: (0, 0))],
        out_specs=[pl.BlockSpec((_ROW_BLK, 2 * D), lambda i: (i, 0)),
                   pl.BlockSpec((_ROW_BLK, D), lambda i: (i, 0))],
        out_shape=[jax.ShapeDtypeStruct((N, 2 * D), jnp.float32),
                   jax.ShapeDtypeStruct((N, D), jnp.float32)],
    )(h, w3)

    outp = _edge_kernel(qv, kk, eidx)

    out = pl.pallas_call(
        _norm_body,
        grid=(N // _ROW_BLK,),
        in_specs=[pl.BlockSpec((_ROW_BLK, DW), lambda i: (i, 0)),
                  pl.BlockSpec((_ROW_BLK, DW), lambda i: (i, 0)),
                  pl.BlockSpec((16, D), lambda i: (0, 0))],
        out_specs=pl.BlockSpec((_ROW_BLK, D), lambda i: (i, 0)),
        out_shape=jax.ShapeDtypeStruct((N, D), jnp.float32),
    )(outp[:N], outp[NP:NP + N], jnp.asarray(_MEXP))
    return out


# speedup vs baseline: 1.4555x; 1.4555x over previous
import functools

import jax
import jax.numpy as jnp
import numpy as np
from jax import lax
from jax.experimental import pallas as pl
from jax.experimental.pallas import tpu as pltpu
from jax.experimental.pallas import tpu_sc as plsc

N = 10000
E = 320000
IN_DIM = 128
HD = 32
NH = 4
D = NH * HD

NC, NS = 2, 16
NW = NC * NS
EPW = E // NW
C = 32
NPAIR = 156
TAIL = EPW - 2 * NPAIR * C
NP = 10240
RPT = NP // NS

_ROW_BLK = 1000


def _qkv_body(h_ref, w_ref, q_ref, k_ref, v_ref):
    r = jnp.dot(h_ref[...], w_ref[...], preferred_element_type=jnp.float32)
    q_ref[...] = r[:, :D]
    k_ref[...] = r[:, D:2 * D]
    v_ref[...] = r[:, 2 * D:]


def _norm_body(p0_ref, p1_ref, d0_ref, d1_ref, mexp_ref, o_ref):
    p = p0_ref[...] + p1_ref[...]
    d = d0_ref[...] + d1_ref[...]
    dc = jnp.dot(d, mexp_ref[...], preferred_element_type=jnp.float32)
    o_ref[...] = jnp.where(dc > 0, p / dc, 0.0)


def _compute_chunk(qb, kb, vb, wb, sb, ngroups):
    zeros16 = jnp.zeros((16,), jnp.float32)
    lanes0 = lax.iota(jnp.int32, 16)
    UNR = 4
    for g in range(ngroups):
        lanes = lanes0 + (g * 16)

        def dot_body(cb, accs):
            a = list(accs)
            cbase = lanes0 + cb * UNR
            for u in range(UNR):
                rot = (cbase + u) & (HD - 1)
                for h in range(NH):
                    col = rot + (h * HD)
                    qc = plsc.load_gather(qb, [lanes, col])
                    kc = plsc.load_gather(kb, [lanes, col])
                    a[h] = a[h] + qc * kc
            return tuple(a)
        acc = lax.fori_loop(0, HD // UNR, dot_body, (zeros16,) * NH)

        ss = []
        for h in range(NH):
            e = acc[h]
            e = jnp.where(e < 0, e * 0.2, e)
            s = jnp.exp(e)
            plsc.store_scatter(sb, [lanes, jnp.full((16,), h, jnp.int32)], s)
            ss.append(s)

        def scale_body(cb, carry):
            cbase = lanes0 + cb * UNR
            for u in range(UNR):
                rot = (cbase + u) & (HD - 1)
                for h in range(NH):
                    col = rot + (h * HD)
                    vc = plsc.load_gather(vb, [lanes, col])
                    plsc.store_scatter(wb, [lanes, col], vc * ss[h])
            return carry
        lax.fori_loop(0, HD // UNR, scale_body, 0)


def _copy_idx(dst_ref, src_ref):
    for j in range(C // 16):
        dst_ref[pl.ds(j * 16, 16)] = src_ref[pl.ds(j * 16, 16)]


def _edge_kernel_body(q_hbm, k_hbm, v_hbm, src_hbm, dst_hbm, outp, denp,
                      srcA, dstA, dscA, qbA, kbA, vbA, wbA, sbA,
                      srcB, dstB, dscB, qbB, kbB, vbB, wbB, sbB,
                      srcT, dstT,
                      out_sp, den_sp,
                      gsemA, gsemB, ssemA, ssemB, isemA, isemB):
    cid = lax.axis_index("c")
    sid = lax.axis_index("s")
    zeros16 = jnp.zeros((16,), jnp.float32)

    def zrow(r, _):
        for j in range(D // 16):
            wbA[r, pl.ds(j * 16, 16)] = zeros16
        sbA[r] = zeros16
        sbB[r] = zeros16
        return 0
    lax.fori_loop(0, C, zrow, 0)

    for b in range(RPT // C):
        r0 = sid * RPT + b * C
        pltpu.sync_copy(wbA, out_sp.at[pl.ds(r0, C)])
        pltpu.sync_copy(sbA, den_sp.at[pl.ds(r0, C)])

    plsc.subcore_barrier()

    ebase = (cid * NS + sid) * EPW

    pltpu.sync_copy(src_hbm.at[pl.ds(ebase, C)], srcA)
    pltpu.sync_copy(dst_hbm.at[pl.ds(ebase, C)], dstA)
    pltpu.async_copy(q_hbm.at[srcA], qbA, gsemA)
    pltpu.async_copy(k_hbm.at[dstA], kbA, gsemA)
    pltpu.async_copy(v_hbm.at[srcA], vbA, gsemA)

    def pair_body(t, _):
        baseB = ebase + (2 * t + 1) * C
        baseA2 = ebase + (2 * t + 2) * C

        pltpu.async_copy(src_hbm.at[pl.ds(baseB, C)], srcB, isemB)
        pltpu.async_copy(dst_hbm.at[pl.ds(baseB, C)], dstB, isemB)

        @pl.when(t > 0)
        def _():
            pltpu.make_async_copy(wbA, out_sp.at[dscA], ssemA).wait()
            pltpu.make_async_copy(sbA, den_sp.at[dscA], ssemA).wait()

        pltpu.make_async_copy(q_hbm.at[srcA], qbA, gsemA).wait()
        pltpu.make_async_copy(k_hbm.at[dstA], kbA, gsemA).wait()
        pltpu.make_async_copy(v_hbm.at[srcA], vbA, gsemA).wait()
        _copy_idx(dscA, dstA)

        @pl.when(t < NPAIR - 1)
        def _():
            pltpu.async_copy(src_hbm.at[pl.ds(baseA2, C)], srcA, isemA)
            pltpu.async_copy(dst_hbm.at[pl.ds(baseA2, C)], dstA, isemA)

        pltpu.make_async_copy(src_hbm.at[pl.ds(baseB, C)], srcB, isemB).wait()
        pltpu.make_async_copy(dst_hbm.at[pl.ds(baseB, C)], dstB, isemB).wait()
        pltpu.async_copy(q_hbm.at[srcB], qbB, gsemB)
        pltpu.async_copy(k_hbm.at[dstB], kbB, gsemB)
        pltpu.async_copy(v_hbm.at[srcB], vbB, gsemB)

        _compute_chunk(qbA, kbA, vbA, wbA, sbA, C // 16)
        pltpu.async_copy(wbA, out_sp.at[dscA], ssemA, add=True)
        pltpu.async_copy(sbA, den_sp.at[dscA], ssemA, add=True)

        @pl.when(t < NPAIR - 1)
        def _():
            pltpu.make_async_copy(src_hbm.at[pl.ds(baseA2, C)], srcA,
                                  isemA).wait()
            pltpu.make_async_copy(dst_hbm.at[pl.ds(baseA2, C)], dstA,
                                  isemA).wait()
            pltpu.async_copy(q_hbm.at[srcA], qbA, gsemA)
            pltpu.async_copy(k_hbm.at[dstA], kbA, gsemA)
            pltpu.async_copy(v_hbm.at[srcA], vbA, gsemA)

        @pl.when(t > 0)
        def _():
            pltpu.make_async_copy(wbB, out_sp.at[dscB], ssemB).wait()
            pltpu.make_async_copy(sbB, den_sp.at[dscB], ssemB).wait()

        pltpu.make_async_copy(q_hbm.at[srcB], qbB, gsemB).wait()
        pltpu.make_async_copy(k_hbm.at[dstB], kbB, gsemB).wait()
        pltpu.make_async_copy(v_hbm.at[srcB], vbB, gsemB).wait()
        _copy_idx(dscB, dstB)
        _compute_chunk(qbB, kbB, vbB, wbB, sbB, C // 16)
        pltpu.async_copy(wbB, out_sp.at[dscB], ssemB, add=True)
        pltpu.async_copy(sbB, den_sp.at[dscB], ssemB, add=True)
        return 0

    lax.fori_loop(0, NPAIR, pair_body, 0)

    pltpu.make_async_copy(wbA, out_sp.at[dscA], ssemA).wait()
    pltpu.make_async_copy(sbA, den_sp.at[dscA], ssemA).wait()
    pltpu.make_async_copy(wbB, out_sp.at[dscB], ssemB).wait()
    pltpu.make_async_copy(sbB, den_sp.at[dscB], ssemB).wait()

    tbase = ebase + 2 * NPAIR * C
    pltpu.sync_copy(src_hbm.at[pl.ds(tbase, TAIL)], srcT)
    pltpu.sync_copy(dst_hbm.at[pl.ds(tbase, TAIL)], dstT)
    pltpu.sync_copy(q_hbm.at[srcT], qbA.at[pl.ds(0, TAIL)])
    pltpu.sync_copy(k_hbm.at[dstT], kbA.at[pl.ds(0, TAIL)])
    pltpu.sync_copy(v_hbm.at[srcT], vbA.at[pl.ds(0, TAIL)])
    _compute_chunk(qbA, kbA, vbA, wbA, sbA, TAIL // 16)
    pltpu.sync_copy(wbA.at[pl.ds(0, TAIL)], out_sp.at[dstT], add=True)
    pltpu.sync_copy(sbA.at[pl.ds(0, TAIL)], den_sp.at[dstT], add=True)

    plsc.subcore_barrier()

    r0 = sid * RPT
    pltpu.sync_copy(out_sp.at[pl.ds(r0, RPT)], outp.at[pl.ds(cid * NP + r0, RPT)])
    pltpu.sync_copy(den_sp.at[pl.ds(r0, RPT)], denp.at[pl.ds(cid * NP + r0, RPT)])


_edge_kernel = functools.partial(
    pl.kernel,
    out_type=(jax.ShapeDtypeStruct((NC * NP, D), jnp.float32),
              jax.ShapeDtypeStruct((NC * NP, 16), jnp.float32)),
    mesh=plsc.VectorSubcoreMesh(core_axis_name="c", subcore_axis_name="s"),
    compiler_params=pltpu.CompilerParams(use_tc_tiling_on_sc=False,
                                         needs_layout_passes=False),
    scratch_types=(
        pltpu.VMEM((C,), jnp.int32),
        pltpu.VMEM((C,), jnp.int32),
        pltpu.VMEM((C,), jnp.int32),
        pltpu.VMEM((C, D), jnp.float32),
        pltpu.VMEM((C, D), jnp.float32),
        pltpu.VMEM((C, D), jnp.float32),
        pltpu.VMEM((C, D), jnp.float32),
        pltpu.VMEM((C, 16), jnp.float32),
        pltpu.VMEM((C,), jnp.int32),
        pltpu.VMEM((C,), jnp.int32),
        pltpu.VMEM((C,), jnp.int32),
        pltpu.VMEM((C, D), jnp.float32),
        pltpu.VMEM((C, D), jnp.float32),
        pltpu.VMEM((C, D), jnp.float32),
        pltpu.VMEM((C, D), jnp.float32),
        pltpu.VMEM((C, 16), jnp.float32),
        pltpu.VMEM((TAIL,), jnp.int32),
        pltpu.VMEM((TAIL,), jnp.int32),
        pltpu.VMEM_SHARED((NP, D), jnp.float32),
        pltpu.VMEM_SHARED((NP, 16), jnp.float32),
        pltpu.SemaphoreType.DMA,
        pltpu.SemaphoreType.DMA,
        pltpu.SemaphoreType.DMA,
        pltpu.SemaphoreType.DMA,
        pltpu.SemaphoreType.DMA,
        pltpu.SemaphoreType.DMA,
    ),
)(_edge_kernel_body)


_MEXP = np.zeros((16, D), np.float32)
for _h in range(NH):
    _MEXP[_h, _h * HD:(_h + 1) * HD] = 1.0


def kernel(h, edge_index, Wq, Wk, Wv):
    h = h.astype(jnp.float32)
    src = edge_index[0].astype(jnp.int32)
    dst = edge_index[1].astype(jnp.int32)
    wq = jnp.transpose(Wq, (1, 0, 2)).reshape(IN_DIM, D)
    wk = jnp.transpose(Wk, (1, 0, 2)).reshape(IN_DIM, D)
    wv = jnp.transpose(Wv, (1, 0, 2)).reshape(IN_DIM, D)
    w3 = jnp.concatenate([wq, wk, wv], axis=1)

    qq, kk, vv = pl.pallas_call(
        _qkv_body,
        grid=(N // _ROW_BLK,),
        in_specs=[pl.BlockSpec((_ROW_BLK, IN_DIM), lambda i: (i, 0)),
                  pl.BlockSpec((IN_DIM, 3 * D), lambda i: (0, 0))],
        out_specs=[pl.BlockSpec((_ROW_BLK, D), lambda i: (i, 0))] * 3,
        out_shape=[jax.ShapeDtypeStruct((N, D), jnp.float32)] * 3,
    )(h, w3)

    outp, denp = _edge_kernel(qq, kk, vv, src, dst)

    out = pl.pallas_call(
        _norm_body,
        grid=(N // _ROW_BLK,),
        in_specs=[pl.BlockSpec((_ROW_BLK, D), lambda i: (i, 0)),
                  pl.BlockSpec((_ROW_BLK, D), lambda i: (i, 0)),
                  pl.BlockSpec((_ROW_BLK, 16), lambda i: (i, 0)),
                  pl.BlockSpec((_ROW_BLK, 16), lambda i: (i, 0)),
                  pl.BlockSpec((16, D), lambda i: (0, 0))],
        out_specs=pl.BlockSpec((_ROW_BLK, D), lambda i: (i, 0)),
        out_shape=jax.ShapeDtypeStruct((N, D), jnp.float32),
    )(outp[:N], outp[NP:NP + N], denp[:N], denp[NP:NP + N], jnp.asarray(_MEXP))
    return out
